# baseline (speedup 1.0000x reference)
import functools

import jax
import jax.numpy as jnp
from jax import lax
from jax.experimental import pallas as pl
from jax.experimental.pallas import tpu as pltpu
from jax.experimental.pallas import tpu_sc as plsc

L_TOK = 65536
D = 7
K = 157
NC, NS, LANES = 2, 16, 16
NW = NC * NS
TPT = L_TOK // NW
G = 128
NG = TPT // G
TPI = 4
OFFS = (0, 16, 32, 48, 64, 80, 96, 112, 128, 141)
EPS2 = 1e-16


def _rsqrt(s):
    i = lax.bitcast_convert_type(s, jnp.int32)
    i = jnp.int32(0x5F3759DF) - (i >> 1)
    r = lax.bitcast_convert_type(i, jnp.float32)
    for _ in range(3):
        r = r * (1.5 - 0.5 * s * r * r)
    return r


def _tsum(vs):
    vs = list(vs)
    while len(vs) > 1:
        nxt = [vs[i] + vs[i + 1] for i in range(0, len(vs) - 1, 2)]
        if len(vs) % 2:
            nxt.append(vs[-1])
        vs = nxt
    return vs[0]


_mesh = plsc.VectorSubcoreMesh(core_axis_name="c", subcore_axis_name="s")


@functools.partial(
    pl.kernel,
    mesh=_mesh,
    out_type=[
        jax.ShapeDtypeStruct((L_TOK, K), jnp.float32),
        jax.ShapeDtypeStruct((L_TOK,), jnp.float32),
        jax.ShapeDtypeStruct((L_TOK,), jnp.int32),
    ],
    compiler_params=pltpu.CompilerParams(needs_layout_passes=False),
    scratch_types=[
        pltpu.VMEM((D * K,), jnp.float32),
        pltpu.VMEM((G * D + LANES * D,), jnp.float32),
        pltpu.VMEM((G, K), jnp.float32),
        pltpu.VMEM((G, K), jnp.float32),
        pltpu.VMEM((G,), jnp.float32),
        pltpu.VMEM((G,), jnp.int32),
        pltpu.SemaphoreType.DMA,
        pltpu.SemaphoreType.DMA,
    ],
)
def _sc_kernel(x_hbm, wt_hbm, cos_hbm, val_hbm, idx_hbm,
               wv, xg, obufA, obufB, vbuf, ibuf, semA, semB):
    wid = lax.axis_index("s") * NC + lax.axis_index("c")
    t0 = wid * TPT
    iota = lax.iota(jnp.int32, LANES)

    pltpu.sync_copy(wt_hbm, wv)
    wn = []
    kvecs = []
    for off in OFFS:
        wd = [wv[pl.ds(d * K + off, LANES)] for d in range(D)]
        s2 = wd[0] * wd[0]
        for d in range(1, D):
            s2 += wd[d] * wd[d]
        r2 = _rsqrt(jnp.maximum(s2, EPS2))
        wn.append([wd[d] * r2 for d in range(D)])
        kvecs.append(iota + off)

    def one_token(obuf, t, r1s):
        xv = xg[pl.ds(t * D, LANES)]
        r1 = jnp.full((LANES,), r1s)
        bxs = [jnp.full((LANES,), xv[d]) * r1 for d in range(D)]
        m = jnp.full((LANES,), -jnp.inf, jnp.float32)
        ib = jnp.zeros((LANES,), jnp.int32)
        for ci, off in enumerate(OFFS):
            cos = _tsum([bxs[d] * wn[ci][d] for d in range(D)])
            obuf[t, pl.ds(off, LANES)] = cos
            upd = cos > m
            m = jnp.maximum(m, cos)
            ib = jnp.where(upd, kvecs[ci], ib)
        rowmax = jnp.max(m)
        cand = jnp.where(m == jnp.full((LANES,), rowmax),
                         ib, jnp.int32(1 << 30))
        return rowmax, jnp.min(cand)

    def run_group(gbase, obuf):
        pltpu.sync_copy(x_hbm.at[pl.ds(gbase * D, G * D)], xg.at[pl.ds(0, G * D)])

        @plsc.parallel_loop(0, G // TPI, 1, unroll=2,
                            carry=(jnp.zeros((LANES,), jnp.float32),
                                   jnp.zeros((LANES,), jnp.int32)))
        def pair(p, carry2):
            vacc, iacc = carry2
            t = p * TPI
            gidx = t * D + iota * D
            gx = [plsc.load_gather(xg, [gidx + d]) for d in range(D)]
            s1v = _tsum([gx[d] * gx[d] for d in range(D)])
            r1v = _rsqrt(jnp.maximum(s1v, EPS2))
            for j in range(TPI):
                rowmax, rowidx = one_token(obuf, t + j, r1v[j])
                sel = iota == (t + j) % LANES
                vacc = jnp.where(sel, rowmax, vacc)
                iacc = jnp.where(sel, rowidx, iacc)

            @pl.when((t + TPI) % LANES == 0)
            def _flush():
                base = (t // LANES) * LANES
                vbuf[pl.ds(base, LANES)] = vacc
                ibuf[pl.ds(base, LANES)] = iacc

            return (vacc, iacc)

        pltpu.sync_copy(vbuf, val_hbm.at[pl.ds(gbase, G)])
        pltpu.sync_copy(ibuf, idx_hbm.at[pl.ds(gbase, G)])

    def group2(g2, carry):
        gbaseA = t0 + (2 * g2) * G
        gbaseB = gbaseA + G

        @pl.when(g2 > 0)
        def _waitA():
            pltpu.make_async_copy(
                obufA, cos_hbm.at[pl.ds(gbaseA, G), :], semA).wait()

        run_group(gbaseA, obufA)
        pltpu.make_async_copy(
            obufA, cos_hbm.at[pl.ds(gbaseA, G), :], semA).start()

        @pl.when(g2 > 0)
        def _waitB():
            pltpu.make_async_copy(
                obufB, cos_hbm.at[pl.ds(gbaseB, G), :], semB).wait()

        run_group(gbaseB, obufB)
        pltpu.make_async_copy(
            obufB, cos_hbm.at[pl.ds(gbaseB, G), :], semB).start()
        return carry

    lax.fori_loop(0, NG // 2, group2, 0)
    pltpu.make_async_copy(obufA, cos_hbm.at[pl.ds(t0, G), :], semA).wait()
    pltpu.make_async_copy(obufB, cos_hbm.at[pl.ds(t0, G), :], semB).wait()


def kernel(embedded_sequence, weights):
    xflat = embedded_sequence.reshape(-1)
    wt = weights.T.reshape(-1)
    cos, vals, idx = _sc_kernel(xflat, wt)
    return (cos.reshape(1, L_TOK, K),
            vals.reshape(1, L_TOK, 1),
            idx.reshape(1, L_TOK, 1))


# speedup vs baseline: 1.1445x; 1.1445x over previous
import functools

import jax
import jax.numpy as jnp
from jax import lax
from jax.experimental import pallas as pl
from jax.experimental.pallas import tpu as pltpu
from jax.experimental.pallas import tpu_sc as plsc

L_TOK = 65536
D = 7
K = 157
NC, NS, LANES = 2, 16, 16
NW = NC * NS
TPT = L_TOK // NW
G = 128
NG = TPT // G
TPI = 4
OFFS = (0, 16, 32, 48, 64, 80, 96, 112, 128, 141)
EPS2 = 1e-16


def _rsqrt(s):
    i = lax.bitcast_convert_type(s, jnp.int32)
    i = jnp.int32(0x5F3759DF) - (i >> 1)
    r = lax.bitcast_convert_type(i, jnp.float32)
    for _ in range(3):
        r = r * (1.5 - 0.5 * s * r * r)
    return r


def _tsum(vs):
    vs = list(vs)
    while len(vs) > 1:
        nxt = [vs[i] + vs[i + 1] for i in range(0, len(vs) - 1, 2)]
        if len(vs) % 2:
            nxt.append(vs[-1])
        vs = nxt
    return vs[0]


_mesh = plsc.VectorSubcoreMesh(core_axis_name="c", subcore_axis_name="s")


@functools.partial(
    pl.kernel,
    mesh=_mesh,
    out_type=[
        jax.ShapeDtypeStruct((L_TOK, K), jnp.float32),
        jax.ShapeDtypeStruct((L_TOK,), jnp.float32),
        jax.ShapeDtypeStruct((L_TOK,), jnp.int32),
    ],
    compiler_params=pltpu.CompilerParams(needs_layout_passes=False),
    scratch_types=[
        pltpu.VMEM((D * K,), jnp.float32),
        pltpu.VMEM((G * D + LANES,), jnp.float32),
        pltpu.VMEM((G * D + LANES,), jnp.float32),
        pltpu.VMEM((G, K), jnp.float32),
        pltpu.VMEM((G, K), jnp.float32),
        pltpu.VMEM((G,), jnp.float32),
        pltpu.VMEM((G,), jnp.int32),
        pltpu.SemaphoreType.DMA,
        pltpu.SemaphoreType.DMA,
        pltpu.SemaphoreType.DMA,
        pltpu.SemaphoreType.DMA,
    ],
)
def _sc_kernel(x_hbm, wt_hbm, cos_hbm, val_hbm, idx_hbm,
               wv, xgA, xgB, obufA, obufB, vbuf, ibuf, semA, semB, sxA, sxB):
    wid = lax.axis_index("s") * NC + lax.axis_index("c")
    t0 = wid * TPT
    iota = lax.iota(jnp.int32, LANES)

    pltpu.sync_copy(wt_hbm, wv)
    wn = []
    kvecs = []
    for off in OFFS:
        wd = [wv[pl.ds(d * K + off, LANES)] for d in range(D)]
        s2 = wd[0] * wd[0]
        for d in range(1, D):
            s2 += wd[d] * wd[d]
        r2 = _rsqrt(jnp.maximum(s2, EPS2))
        wn.append([wd[d] * r2 for d in range(D)])
        kvecs.append(iota + off)

    def one_token(obuf, xg, t):
        xv = xg[pl.ds(t * D, LANES)]
        bx = [jnp.full((LANES,), xv[d]) for d in range(D)]
        s1 = _tsum([bx[d] * bx[d] for d in range(D)])
        r1 = _rsqrt(jnp.maximum(s1, EPS2))
        bxs = [bx[d] * r1 for d in range(D)]
        m = jnp.full((LANES,), -jnp.inf, jnp.float32)
        ib = jnp.zeros((LANES,), jnp.int32)
        for ci, off in enumerate(OFFS):
            cos = _tsum([bxs[d] * wn[ci][d] for d in range(D)])
            obuf[t, pl.ds(off, LANES)] = cos
            upd = cos > m
            m = jnp.maximum(m, cos)
            ib = jnp.where(upd, kvecs[ci], ib)
        rowmax = jnp.max(m)
        cand = jnp.where(m == jnp.full((LANES,), rowmax),
                         ib, jnp.int32(1 << 30))
        return rowmax, jnp.min(cand)

    def _xslice(gbase):
        base = jnp.minimum(gbase, L_TOK - G) * D
        return x_hbm.at[pl.ds(base, G * D)]

    def run_group(gbase, obuf, xg):

        @plsc.parallel_loop(0, G // TPI, 1, unroll=2,
                            carry=(jnp.zeros((LANES,), jnp.float32),
                                   jnp.zeros((LANES,), jnp.int32)))
        def pair(p, carry2):
            vacc, iacc = carry2
            t = p * TPI
            for j in range(TPI):
                rowmax, rowidx = one_token(obuf, xg, t + j)
                sel = iota == (t + j) % LANES
                vacc = jnp.where(sel, rowmax, vacc)
                iacc = jnp.where(sel, rowidx, iacc)

            @pl.when((t + TPI) % LANES == 0)
            def _flush():
                base = (t // LANES) * LANES
                vbuf[pl.ds(base, LANES)] = vacc
                ibuf[pl.ds(base, LANES)] = iacc

            return (vacc, iacc)

        pltpu.sync_copy(vbuf, val_hbm.at[pl.ds(gbase, G)])
        pltpu.sync_copy(ibuf, idx_hbm.at[pl.ds(gbase, G)])

    def group2(g2, carry):
        gbaseA = t0 + (2 * g2) * G
        gbaseB = gbaseA + G

        pltpu.make_async_copy(_xslice(gbaseA), xgA.at[pl.ds(0, G * D)], sxA).wait()
        pltpu.make_async_copy(_xslice(gbaseB), xgB.at[pl.ds(0, G * D)], sxB).start()

        @pl.when(g2 > 0)
        def _waitA():
            pltpu.make_async_copy(
                obufA, cos_hbm.at[pl.ds(gbaseA, G), :], semA).wait()

        run_group(gbaseA, obufA, xgA)
        pltpu.make_async_copy(
            obufA, cos_hbm.at[pl.ds(gbaseA, G), :], semA).start()
        pltpu.make_async_copy(
            _xslice(gbaseA + 2 * G), xgA.at[pl.ds(0, G * D)], sxA).start()

        pltpu.make_async_copy(_xslice(gbaseB), xgB.at[pl.ds(0, G * D)], sxB).wait()

        @pl.when(g2 > 0)
        def _waitB():
            pltpu.make_async_copy(
                obufB, cos_hbm.at[pl.ds(gbaseB, G), :], semB).wait()

        run_group(gbaseB, obufB, xgB)
        pltpu.make_async_copy(
            obufB, cos_hbm.at[pl.ds(gbaseB, G), :], semB).start()
        return carry

    pltpu.make_async_copy(_xslice(t0), xgA.at[pl.ds(0, G * D)], sxA).start()
    lax.fori_loop(0, NG // 2, group2, 0)
    pltpu.make_async_copy(_xslice(t0), xgA.at[pl.ds(0, G * D)], sxA).wait()
    pltpu.make_async_copy(obufA, cos_hbm.at[pl.ds(t0, G), :], semA).wait()
    pltpu.make_async_copy(obufB, cos_hbm.at[pl.ds(t0, G), :], semB).wait()


def kernel(embedded_sequence, weights):
    xflat = embedded_sequence.reshape(-1)
    wt = weights.T.reshape(-1)
    cos, vals, idx = _sc_kernel(xflat, wt)
    return (cos.reshape(1, L_TOK, K),
            vals.reshape(1, L_TOK, 1),
            idx.reshape(1, L_TOK, 1))
